# baseline (speedup 1.0000x reference)
import functools

import jax
import jax.numpy as jnp
from jax import lax
from jax.experimental import pallas as pl
from jax.experimental.pallas import tpu as pltpu
from jax.experimental.pallas import tpu_sc as plsc

_NC = 2
_NS = 16
_NW = _NC * _NS

E = 64
H = 1024
F = 512
T = 8192
K = 2
N = T * K
B = 128
P = N + E * B
G = P // B


def _sc_metadata(order, ids, w_flat, pstart, start, src_init):
    mesh = plsc.VectorSubcoreMesh(core_axis_name="c", subcore_axis_name="s")

    @functools.partial(
        pl.kernel,
        mesh=mesh,
        compiler_params=pltpu.CompilerParams(needs_layout_passes=False),
        out_type=(
            jax.ShapeDtypeStruct((P,), jnp.int32),
            jax.ShapeDtypeStruct((P,), jnp.float32),
            jax.ShapeDtypeStruct((N,), jnp.int32),
        ),
        scratch_types=[
            pltpu.VMEM((N,), jnp.int32),
            pltpu.VMEM((N,), jnp.int32),
            pltpu.VMEM((N,), jnp.float32),
            pltpu.VMEM((E,), jnp.int32),
            pltpu.VMEM((E,), jnp.int32),
            pltpu.VMEM((P,), jnp.int32),
            pltpu.VMEM((P,), jnp.float32),
            pltpu.VMEM((N,), jnp.int32),
        ],
    )
    def meta_kernel(order_h, ids_h, wflat_h, pstart_h, start_h, srcinit_h,
                    srctok_h, wslot_h, slotkt_h,
                    order_v, ids_v, wflat_v, pstart_v, start_v,
                    srctok_v, wslot_v, slotkt_v):
        wid = lax.axis_index("s") * _NC + lax.axis_index("c")

        @pl.when(wid == 0)
        def _():
            pltpu.sync_copy(order_h, order_v)
            pltpu.sync_copy(ids_h, ids_v)
            pltpu.sync_copy(wflat_h, wflat_v)
            pltpu.sync_copy(pstart_h, pstart_v)
            pltpu.sync_copy(start_h, start_v)
            pltpu.sync_copy(srcinit_h, srctok_v)
            lanes = lax.iota(jnp.int32, 16)

            def body(j, carry):
                base = j * 16
                ordv = order_v[pl.ds(base, 16)]
                sid = plsc.load_gather(ids_v, [ordv])
                ps = plsc.load_gather(pstart_v, [sid])
                st = plsc.load_gather(start_v, [sid])
                slot = ps + base + lanes - st
                tok = jnp.right_shift(ordv, 1)
                plsc.store_scatter(srctok_v, [slot], tok)
                wgt = plsc.load_gather(wflat_v, [ordv])
                plsc.store_scatter(wslot_v, [slot], wgt)
                q = jnp.left_shift(jnp.bitwise_and(ordv, 1), 13) + tok
                plsc.store_scatter(slotkt_v, [q], slot)
                return carry

            lax.fori_loop(0, N // 16, body, 0)
            pltpu.sync_copy(srctok_v, srctok_h)
            pltpu.sync_copy(wslot_v, wslot_h)
            pltpu.sync_copy(slotkt_v, slotkt_h)

    return meta_kernel(order, ids, w_flat, pstart, start, src_init)


def _routing_metadata(top_experts, expert_weights):
    ids = top_experts.reshape(-1).astype(jnp.int32)
    order = jnp.argsort(ids).astype(jnp.int32)
    counts = jnp.bincount(ids, length=E)
    padded = ((counts + B - 1) // B) * B
    pstart = (jnp.cumsum(padded) - padded).astype(jnp.int32)
    start = (jnp.cumsum(counts) - counts).astype(jnp.int32)
    src_init = jnp.arange(P, dtype=jnp.int32) % T
    w_flat = expert_weights.reshape(-1)
    src_tok, w_slot, slot_kt = _sc_metadata(
        order, ids, w_flat, pstart, start, src_init)
    be0 = jnp.zeros((G,), jnp.int32).at[pstart // B].max(
        jnp.arange(E, dtype=jnp.int32))
    block_expert = lax.cummax(be0)
    total = jnp.sum(padded)
    grow = jnp.arange(G, dtype=jnp.int32) * B
    block_valid = (grow < total).astype(jnp.int32)
    return src_tok, w_slot, slot_kt, block_expert, block_valid


def _sc_gather_rows(table, idx, n_rows, ch):
    rows_w = n_rows // _NW
    nch = rows_w // ch
    nbuf = 3
    mesh = plsc.VectorSubcoreMesh(core_axis_name="c", subcore_axis_name="s")

    @functools.partial(
        pl.kernel,
        mesh=mesh,
        out_type=jax.ShapeDtypeStruct((n_rows, H), jnp.float32),
        scratch_types=[
            pltpu.VMEM((rows_w,), jnp.int32),
        ]
        + [pltpu.VMEM((ch, H), jnp.float32) for _ in range(nbuf)]
        + [pltpu.SemaphoreType.DMA for _ in range(2 * nbuf)],
    )
    def gather_kernel(table_hbm, idx_hbm, out_hbm, idx_v, *scratch):
        bufs = scratch[:nbuf]
        gsems = scratch[nbuf:2 * nbuf]
        ssems = scratch[2 * nbuf:]
        wid = lax.axis_index("s") * _NC + lax.axis_index("c")
        base = wid * rows_w
        pltpu.sync_copy(idx_hbm.at[pl.ds(base, rows_w)], idx_v)

        def start_gather(i):
            b = i % nbuf
            return pltpu.async_copy(
                table_hbm.at[idx_v.at[pl.ds(i * ch, ch)]], bufs[b], gsems[b])

        def start_store(i):
            b = i % nbuf
            return pltpu.async_copy(
                bufs[b], out_hbm.at[pl.ds(base + i * ch, ch)], ssems[b])

        gathers = {}
        stores = {}
        for i in range(min(nbuf, nch)):
            gathers[i] = start_gather(i)
        for i in range(nch):
            gathers.pop(i).wait()
            stores[i] = start_store(i)
            j = i + nbuf
            if j < nch:
                stores.pop(j - nbuf).wait()
                gathers[j] = start_gather(j)
        for i in sorted(stores):
            stores.pop(i).wait()

    return gather_kernel(table, idx)


def _grouped_gemm(x_half, w1, w2, w_slot_half, be_half, bv_half, off, y_prev):
    G_h = x_half.shape[0] // B

    def body(be_ref, bv_ref, *refs):
        y_ref = refs[-1]
        x_ref, w1_ref, w2_ref, ws_ref = refs[:4]
        g = pl.program_id(0)

        @pl.when(bv_ref[g] == 1)
        def _():
            x = x_ref[...]
            h = jnp.dot(x, w1_ref[0], preferred_element_type=jnp.float32)
            gate = h[:, :F]
            up = h[:, F:]
            act = (gate * jax.nn.sigmoid(gate)) * up
            y = jnp.dot(act, w2_ref[0], preferred_element_type=jnp.float32)
            y_ref[...] = y * ws_ref[0, 0, :][:, None]

    in_specs = [
        pl.BlockSpec((B, H), lambda g, be, bv: (g, 0)),
        pl.BlockSpec((1, H, 2 * F), lambda g, be, bv: (be[g], 0, 0)),
        pl.BlockSpec((1, F, H), lambda g, be, bv: (be[g], 0, 0)),
        pl.BlockSpec((1, 1, B), lambda g, be, bv: (g, 0, 0)),
    ]
    args = [be_half, bv_half, x_half, w1, w2, w_slot_half.reshape(G_h, 1, B)]
    io_aliases = {}
    if y_prev is not None:
        in_specs.append(pl.BlockSpec((B, H), lambda g, be, bv: (0, 0)))
        args.append(y_prev)
        io_aliases = {6: 0}

    grid_spec = pltpu.PrefetchScalarGridSpec(
        num_scalar_prefetch=2,
        grid=(G_h,),
        in_specs=in_specs,
        out_specs=pl.BlockSpec((B, H), lambda g, be, bv: (g + off, 0)),
    )
    return pl.pallas_call(
        body,
        grid_spec=grid_spec,
        out_shape=jax.ShapeDtypeStruct((P, H), jnp.float32),
        input_output_aliases=io_aliases,
    )(*args)


def _pairsum(zz):
    BT = 256

    def body(a_ref, b_ref, o_ref):
        o_ref[...] = a_ref[...] + b_ref[...]

    return pl.pallas_call(
        body,
        grid=(T // BT,),
        in_specs=[
            pl.BlockSpec((BT, H), lambda i: (i, 0)),
            pl.BlockSpec((BT, H), lambda i: (i + T // BT, 0)),
        ],
        out_specs=pl.BlockSpec((BT, H), lambda i: (i, 0)),
        out_shape=jax.ShapeDtypeStruct((T, H), jnp.float32),
    )(zz, zz)


def kernel(hidden_states, expert_weights, top_experts, w1, w2):
    src_tok, w_slot, slot_kt, block_expert, block_valid = _routing_metadata(
        top_experts, expert_weights)
    P1, G1 = P // 2, G // 2
    x1 = _sc_gather_rows(hidden_states, src_tok[:P1], P1, ch=32)
    x2 = _sc_gather_rows(hidden_states, src_tok[P1:], P1, ch=32)
    y0 = _grouped_gemm(x1, w1, w2, w_slot[:P1], block_expert[:G1],
                       block_valid[:G1], off=0, y_prev=None)
    y_padded = _grouped_gemm(x2, w1, w2, w_slot[P1:], block_expert[G1:],
                             block_valid[G1:], off=G1, y_prev=y0)
    zz = _sc_gather_rows(y_padded, slot_kt, N, ch=32)
    return _pairsum(zz)


# speedup vs baseline: 1.0186x; 1.0186x over previous
import functools

import jax
import jax.numpy as jnp
from jax import lax
from jax.experimental import pallas as pl
from jax.experimental.pallas import tpu as pltpu
from jax.experimental.pallas import tpu_sc as plsc

_NC = 2
_NS = 16
_NW = _NC * _NS

E = 64
H = 1024
F = 512
T = 8192
K = 2
N = T * K
B = 128
P = N + E * B
G = P // B


def _sc_metadata(order, ids, w_flat, pstart, start, src_init):
    mesh = plsc.VectorSubcoreMesh(core_axis_name="c", subcore_axis_name="s")

    @functools.partial(
        pl.kernel,
        mesh=mesh,
        compiler_params=pltpu.CompilerParams(needs_layout_passes=False),
        out_type=(
            jax.ShapeDtypeStruct((P,), jnp.int32),
            jax.ShapeDtypeStruct((P,), jnp.float32),
            jax.ShapeDtypeStruct((N,), jnp.int32),
        ),
        scratch_types=[
            pltpu.VMEM((N,), jnp.int32),
            pltpu.VMEM((N,), jnp.int32),
            pltpu.VMEM((N,), jnp.float32),
            pltpu.VMEM((E,), jnp.int32),
            pltpu.VMEM((E,), jnp.int32),
            pltpu.VMEM((P,), jnp.int32),
            pltpu.VMEM((P,), jnp.float32),
            pltpu.VMEM((N,), jnp.int32),
        ],
    )
    def meta_kernel(order_h, ids_h, wflat_h, pstart_h, start_h, srcinit_h,
                    srctok_h, wslot_h, slotkt_h,
                    order_v, ids_v, wflat_v, pstart_v, start_v,
                    srctok_v, wslot_v, slotkt_v):
        wid = lax.axis_index("s") * _NC + lax.axis_index("c")

        @pl.when(wid == 0)
        def _():
            pltpu.sync_copy(order_h, order_v)
            pltpu.sync_copy(ids_h, ids_v)
            pltpu.sync_copy(wflat_h, wflat_v)
            pltpu.sync_copy(pstart_h, pstart_v)
            pltpu.sync_copy(start_h, start_v)
            pltpu.sync_copy(srcinit_h, srctok_v)
            lanes = lax.iota(jnp.int32, 16)

            def body(j, carry):
                base = j * 16
                ordv = order_v[pl.ds(base, 16)]
                sid = plsc.load_gather(ids_v, [ordv])
                ps = plsc.load_gather(pstart_v, [sid])
                st = plsc.load_gather(start_v, [sid])
                slot = ps + base + lanes - st
                tok = jnp.right_shift(ordv, 1)
                plsc.store_scatter(srctok_v, [slot], tok)
                wgt = plsc.load_gather(wflat_v, [ordv])
                plsc.store_scatter(wslot_v, [slot], wgt)
                q = jnp.left_shift(jnp.bitwise_and(ordv, 1), 13) + tok
                plsc.store_scatter(slotkt_v, [q], slot)
                return carry

            lax.fori_loop(0, N // 16, body, 0)
            pltpu.sync_copy(srctok_v, srctok_h)
            pltpu.sync_copy(wslot_v, wslot_h)
            pltpu.sync_copy(slotkt_v, slotkt_h)

    return meta_kernel(order, ids, w_flat, pstart, start, src_init)


def _routing_metadata(top_experts, expert_weights):
    ids = top_experts.reshape(-1).astype(jnp.int32)
    order = jnp.argsort(ids).astype(jnp.int32)
    counts = jnp.bincount(ids, length=E)
    padded = ((counts + B - 1) // B) * B
    pstart = (jnp.cumsum(padded) - padded).astype(jnp.int32)
    start = (jnp.cumsum(counts) - counts).astype(jnp.int32)
    src_init = jnp.arange(P, dtype=jnp.int32) % T
    w_flat = expert_weights.reshape(-1)
    src_tok, w_slot, slot_kt = _sc_metadata(
        order, ids, w_flat, pstart, start, src_init)
    be0 = jnp.zeros((G,), jnp.int32).at[pstart // B].max(
        jnp.arange(E, dtype=jnp.int32))
    block_expert = lax.cummax(be0)
    total = jnp.sum(padded)
    grow = jnp.arange(G, dtype=jnp.int32) * B
    block_valid = (grow < total).astype(jnp.int32)
    return src_tok, w_slot, slot_kt, block_expert, block_valid


def _sc_gather_rows(table, idx, n_rows, ch):
    rows_w = n_rows // _NW
    nch = rows_w // ch
    nbuf = 3
    mesh = plsc.VectorSubcoreMesh(core_axis_name="c", subcore_axis_name="s")

    @functools.partial(
        pl.kernel,
        mesh=mesh,
        out_type=jax.ShapeDtypeStruct((n_rows, H), jnp.float32),
        scratch_types=[
            pltpu.VMEM((rows_w,), jnp.int32),
        ]
        + [pltpu.VMEM((ch, H), jnp.float32) for _ in range(nbuf)]
        + [pltpu.SemaphoreType.DMA for _ in range(2 * nbuf)],
    )
    def gather_kernel(table_hbm, idx_hbm, out_hbm, idx_v, *scratch):
        bufs = scratch[:nbuf]
        gsems = scratch[nbuf:2 * nbuf]
        ssems = scratch[2 * nbuf:]
        wid = lax.axis_index("s") * _NC + lax.axis_index("c")
        base = wid * rows_w
        pltpu.sync_copy(idx_hbm.at[pl.ds(base, rows_w)], idx_v)

        def start_gather(i):
            b = i % nbuf
            return pltpu.async_copy(
                table_hbm.at[idx_v.at[pl.ds(i * ch, ch)]], bufs[b], gsems[b])

        def start_store(i):
            b = i % nbuf
            return pltpu.async_copy(
                bufs[b], out_hbm.at[pl.ds(base + i * ch, ch)], ssems[b])

        gathers = {}
        stores = {}
        for i in range(min(nbuf, nch)):
            gathers[i] = start_gather(i)
        for i in range(nch):
            gathers.pop(i).wait()
            stores[i] = start_store(i)
            j = i + nbuf
            if j < nch:
                stores.pop(j - nbuf).wait()
                gathers[j] = start_gather(j)
        for i in sorted(stores):
            stores.pop(i).wait()

    return gather_kernel(table, idx)


def _grouped_gemm(x_padded, w1, w2, w_slot, block_expert, block_valid):
    def body(be_ref, bv_ref, x_ref, w1_ref, w2_ref, ws_ref, y_ref):
        g = pl.program_id(0)

        @pl.when(bv_ref[g] == 1)
        def _():
            x = x_ref[...]
            h = jnp.dot(x, w1_ref[0], preferred_element_type=jnp.float32)
            gate = h[:, :F]
            up = h[:, F:]
            act = (gate * jax.nn.sigmoid(gate)) * up
            y = jnp.dot(act, w2_ref[0], preferred_element_type=jnp.float32)
            y_ref[...] = y * ws_ref[0, 0, :][:, None]

    grid_spec = pltpu.PrefetchScalarGridSpec(
        num_scalar_prefetch=2,
        grid=(G,),
        in_specs=[
            pl.BlockSpec((B, H), lambda g, be, bv: (g, 0)),
            pl.BlockSpec((1, H, 2 * F), lambda g, be, bv: (be[g], 0, 0)),
            pl.BlockSpec((1, F, H), lambda g, be, bv: (be[g], 0, 0)),
            pl.BlockSpec((1, 1, B), lambda g, be, bv: (g, 0, 0)),
        ],
        out_specs=pl.BlockSpec((B, H), lambda g, be, bv: (g, 0)),
    )
    return pl.pallas_call(
        body,
        grid_spec=grid_spec,
        out_shape=jax.ShapeDtypeStruct((P, H), jnp.float32),
    )(block_expert, block_valid, x_padded, w1, w2, w_slot.reshape(G, 1, B))


def _pairsum(zz):
    BT = 256

    def body(a_ref, b_ref, o_ref):
        o_ref[...] = a_ref[...] + b_ref[...]

    return pl.pallas_call(
        body,
        grid=(T // BT,),
        in_specs=[
            pl.BlockSpec((BT, H), lambda i: (i, 0)),
            pl.BlockSpec((BT, H), lambda i: (i + T // BT, 0)),
        ],
        out_specs=pl.BlockSpec((BT, H), lambda i: (i, 0)),
        out_shape=jax.ShapeDtypeStruct((T, H), jnp.float32),
    )(zz, zz)


def kernel(hidden_states, expert_weights, top_experts, w1, w2):
    src_tok, w_slot, slot_kt, block_expert, block_valid = _routing_metadata(
        top_experts, expert_weights)
    x_padded = _sc_gather_rows(hidden_states, src_tok, P, ch=32)
    y_padded = _grouped_gemm(x_padded, w1, w2, w_slot, block_expert, block_valid)
    zz = _sc_gather_rows(y_padded, slot_kt, N, ch=32)
    return _pairsum(zz)
